# Initial kernel scaffold; baseline (speedup 1.0000x reference)
import functools

import jax
import jax.numpy as jnp
from jax import lax
from jax.experimental import pallas as pl
from jax.experimental.pallas import tpu as pltpu
from jax.experimental.pallas import tpu_sc as plsc

NC = 2
NS = 16
NW = NC * NS
LANES = 16
EB = 80
RB = 1280
F32 = jnp.float32
HIGH = lax.Precision.HIGHEST


def _mesh():
    return plsc.VectorSubcoreMesh(
        core_axis_name="c", subcore_axis_name="s", num_cores=NC, num_subcores=NS
    )


def _sc_deg_body(ei_hbm, out_hbm, idx_v, deg_v):
    c = lax.axis_index("c")
    s = lax.axis_index("s")
    wid = c * NS + s
    npad = deg_v.shape[0]

    zeros16 = jnp.zeros((LANES,), F32)
    def zero_body(i, carry):
        deg_v[pl.ds(i * LANES, LANES)] = zeros16
        return carry
    lax.fori_loop(0, npad // LANES, zero_body, 0)

    e = ei_hbm.shape[1]
    ept = e // NW
    base = wid * ept
    ones16 = jnp.ones((LANES,), F32)

    def body(j, carry):
        pltpu.sync_copy(ei_hbm.at[1, pl.ds(base + j * EB, EB)], idx_v)
        for k in range(EB // LANES):
            d = idx_v[pl.ds(k * LANES, LANES)]
            plsc.addupdate_scatter(deg_v, [d], ones16)
        return carry
    lax.fori_loop(0, ept // EB, body, 0)

    pltpu.sync_copy(deg_v, out_hbm.at[wid])


def _sc_agg_body(m_hbm, ei_hbm, out_hbm, idxs_v, idxd_v, rows_v, zbuf_v, sem,
                 acc_sh):
    c = lax.axis_index("c")
    s = lax.axis_index("s")
    npad = m_hbm.shape[0]
    rpt = npad // NS
    row0 = s * rpt

    zeros16 = jnp.zeros((LANES,), F32)
    def zero_body(i, carry):
        for k in range(128 // LANES):
            zbuf_v[i, pl.ds(k * LANES, LANES)] = zeros16
        return carry
    lax.fori_loop(0, 128, zero_body, 0)
    for q in range(rpt // 128):
        pltpu.sync_copy(zbuf_v, acc_sh.at[pl.ds(row0 + q * 128, 128)])
    plsc.subcore_barrier()

    e = ei_hbm.shape[1]
    ept = e // NW
    base = (c * NS + s) * ept

    def body(j, carry):
        off = base + j * EB
        pltpu.sync_copy(ei_hbm.at[0, pl.ds(off, EB)], idxs_v)
        pltpu.sync_copy(ei_hbm.at[1, pl.ds(off, EB)], idxd_v)
        pltpu.async_copy(m_hbm.at[idxs_v], rows_v, sem).wait()
        pltpu.sync_copy(rows_v, acc_sh.at[idxd_v], add=True)
        return carry
    lax.fori_loop(0, ept // EB, body, 0)
    plsc.subcore_barrier()

    for q in range(rpt // 128):
        r = row0 + q * 128
        pltpu.sync_copy(acc_sh.at[pl.ds(r, 128)], zbuf_v)
        pltpu.sync_copy(zbuf_v, out_hbm.at[c, pl.ds(r, 128)])


def _sc_deg(ei, npad):
    fn = pl.kernel(
        _sc_deg_body,
        out_type=jax.ShapeDtypeStruct((NW, npad), F32),
        mesh=_mesh(),
        scratch_types=[
            pltpu.VMEM((EB,), jnp.int32),
            pltpu.VMEM((npad,), F32),
        ],
    )
    return fn(ei)


def _sc_agg(m, ei, h):
    npad = m.shape[0]
    fn = pl.kernel(
        _sc_agg_body,
        out_type=jax.ShapeDtypeStruct((NC, npad, h), F32),
        mesh=_mesh(),
        scratch_types=[
            pltpu.VMEM((EB,), jnp.int32),
            pltpu.VMEM((EB,), jnp.int32),
            pltpu.VMEM((EB, 128), F32),
            pltpu.VMEM((128, 128), F32),
            pltpu.SemaphoreType.DMA,
            pltpu.VMEM_SHARED((npad, 128), F32),
        ],
    )
    return fn(m, ei)


def _tc0_body(degp_ref, x_ref, w_ref, dinv_ref, m_ref):
    deg = jnp.sum(degp_ref[...], axis=0) + 1.0
    dinv = lax.rsqrt(deg)[:, None]
    dinv_ref[...] = dinv
    t = jnp.dot(x_ref[...], w_ref[...], precision=HIGH,
                preferred_element_type=F32)
    m_ref[...] = t * dinv


def _tc_layer_body(p_ref, m_ref, dinv_ref, b_ref, w_ref, out_ref):
    dinv = dinv_ref[...]
    h = jnp.maximum((p_ref[0] + p_ref[1] + m_ref[...]) * dinv + b_ref[...],
                    0.0)
    t = jnp.dot(h, w_ref[...], precision=HIGH, preferred_element_type=F32)
    out_ref[...] = t * dinv


def _tc_final_body(p_ref, m_ref, dinv_ref, b_ref, batch_ref, wl_ref, bl_ref,
                   out_ref, pool_s, cnt_s):
    i = pl.program_id(0)

    @pl.when(i == 0)
    def _init():
        pool_s[...] = jnp.zeros_like(pool_s)
        cnt_s[...] = jnp.zeros_like(cnt_s)

    dinv = dinv_ref[...]
    h3 = (p_ref[0] + p_ref[1] + m_ref[...]) * dinv + b_ref[...]
    g = lax.broadcasted_iota(jnp.int32, (1, 128), 1)
    oh = (batch_ref[...] == g).astype(F32)
    pool_s[...] += lax.dot_general(oh, h3, (((0,), (0,)), ((), ())),
                                   precision=HIGH, preferred_element_type=F32)
    cnt_s[...] += jnp.sum(oh, axis=0)[None, :]

    @pl.when(i == pl.num_programs(0) - 1)
    def _fin():
        cnt = jnp.maximum(cnt_s[...], 1.0)
        pooled = pool_s[...][:64] / cnt[0, :64][:, None]
        out_ref[...] = jnp.dot(pooled, wl_ref[...], precision=HIGH,
                               preferred_element_type=F32) + bl_ref[...]


def _tc0(degp, xp, w1):
    npad, f = xp.shape
    h = w1.shape[1]
    return pl.pallas_call(
        _tc0_body,
        grid=(npad // RB,),
        in_specs=[
            pl.BlockSpec((NW, RB), lambda i: (0, i)),
            pl.BlockSpec((RB, f), lambda i: (i, 0)),
            pl.BlockSpec((f, h), lambda i: (0, 0)),
        ],
        out_specs=[
            pl.BlockSpec((RB, 1), lambda i: (i, 0)),
            pl.BlockSpec((RB, h), lambda i: (i, 0)),
        ],
        out_shape=[
            jax.ShapeDtypeStruct((npad, 1), F32),
            jax.ShapeDtypeStruct((npad, h), F32),
        ],
    )(degp, xp, w1)


def _tc_layer(p, m, dinv, b, w):
    npad, h = m.shape
    return pl.pallas_call(
        _tc_layer_body,
        grid=(npad // RB,),
        in_specs=[
            pl.BlockSpec((NC, RB, h), lambda i: (0, i, 0)),
            pl.BlockSpec((RB, h), lambda i: (i, 0)),
            pl.BlockSpec((RB, 1), lambda i: (i, 0)),
            pl.BlockSpec((1, h), lambda i: (0, 0)),
            pl.BlockSpec((h, h), lambda i: (0, 0)),
        ],
        out_specs=pl.BlockSpec((RB, h), lambda i: (i, 0)),
        out_shape=jax.ShapeDtypeStruct((npad, h), F32),
    )(p, m, dinv, b, w)


def _tc_final(p, m, dinv, b, batch2d, wlp, blp):
    npad, h = m.shape
    return pl.pallas_call(
        _tc_final_body,
        grid=(npad // RB,),
        in_specs=[
            pl.BlockSpec((NC, RB, h), lambda i: (0, i, 0)),
            pl.BlockSpec((RB, h), lambda i: (i, 0)),
            pl.BlockSpec((RB, 1), lambda i: (i, 0)),
            pl.BlockSpec((1, h), lambda i: (0, 0)),
            pl.BlockSpec((RB, 1), lambda i: (i, 0)),
            pl.BlockSpec((h, 128), lambda i: (0, 0)),
            pl.BlockSpec((1, 128), lambda i: (0, 0)),
        ],
        out_specs=pl.BlockSpec((64, 128), lambda i: (0, 0)),
        out_shape=jax.ShapeDtypeStruct((64, 128), F32),
        scratch_shapes=[
            pltpu.VMEM((128, 128), F32),
            pltpu.VMEM((1, 128), F32),
        ],
    )(p, m, dinv, b, batch2d, wlp, blp)


def kernel(x, edge_index, batch, W1, b1, W2, b2, W3, b3, Wl, bl):
    n, f = x.shape
    h = W1.shape[1]
    c = Wl.shape[1]
    e = edge_index.shape[1]
    assert e % (NW * EB) == 0
    npad = ((n + RB - 1) // RB) * RB

    xp = jnp.pad(x, ((0, npad - n), (0, 0)))
    batch2d = jnp.pad(batch, (0, npad - n), constant_values=127)[:, None]
    b1r = b1[None, :]
    b2r = b2[None, :]
    b3r = b3[None, :]
    wlp = jnp.pad(Wl, ((0, 0), (0, 128 - c)))
    blp = jnp.pad(bl, (0, 128 - c))[None, :]

    degp = _sc_deg(edge_index, npad)
    dinv, m1 = _tc0(degp, xp, W1)
    p1 = _sc_agg(m1, edge_index, h)
    m2 = _tc_layer(p1, m1, dinv, b1r, W2)
    p2 = _sc_agg(m2, edge_index, h)
    m3 = _tc_layer(p2, m2, dinv, b2r, W3)
    p3 = _sc_agg(m3, edge_index, h)
    out = _tc_final(p3, m3, dinv, b3r, batch2d, wlp, blp)
    return out[:, :c]


# speedup vs baseline: 11.5715x; 11.5715x over previous
import functools

import jax
import jax.numpy as jnp
from jax import lax
from jax.experimental import pallas as pl
from jax.experimental.pallas import tpu as pltpu
from jax.experimental.pallas import tpu_sc as plsc

NC = 2
NS = 16
NW = NC * NS
LANES = 16
EB = 80
RB = 1280
F32 = jnp.float32
HIGH = lax.Precision.HIGHEST


def _mesh():
    return plsc.VectorSubcoreMesh(
        core_axis_name="c", subcore_axis_name="s", num_cores=NC, num_subcores=NS
    )


def _sc_deg_body(dst_hbm, out_hbm, idx_v, deg_v):
    c = lax.axis_index("c")
    s = lax.axis_index("s")
    wid = c * NS + s
    npad = deg_v.shape[0]

    zeros16 = jnp.zeros((LANES,), F32)
    def zero_body(i, carry):
        deg_v[pl.ds(i * LANES, LANES)] = zeros16
        return carry
    lax.fori_loop(0, npad // LANES, zero_body, 0)

    e = dst_hbm.shape[0]
    ept = e // NW
    base = wid * ept
    ones16 = jnp.ones((LANES,), F32)

    def body(j, carry):
        pltpu.sync_copy(dst_hbm.at[pl.ds(base + j * EB, EB)], idx_v)
        for k in range(EB // LANES):
            d = idx_v[pl.ds(k * LANES, LANES)]
            plsc.addupdate_scatter(deg_v, [d], ones16)
        return carry
    lax.fori_loop(0, ept // EB, body, 0)

    pltpu.sync_copy(deg_v, out_hbm.at[wid])


def _sc_agg_body(m_hbm, src_hbm, dst_hbm, out_hbm, idxs_v, idxd_v, rows_v,
                 zbuf_v, sem, acc_sh):
    c = lax.axis_index("c")
    s = lax.axis_index("s")
    npad = m_hbm.shape[0]
    rpt = npad // NS
    row0 = s * rpt

    zeros16 = jnp.zeros((LANES,), F32)
    def zero_body(i, carry):
        for k in range(128 // LANES):
            zbuf_v[i, pl.ds(k * LANES, LANES)] = zeros16
        return carry
    lax.fori_loop(0, 128, zero_body, 0)
    for q in range(rpt // 128):
        pltpu.sync_copy(zbuf_v, acc_sh.at[pl.ds(row0 + q * 128, 128)])
    plsc.subcore_barrier()

    e = src_hbm.shape[0]
    ept = e // NW
    base = (c * NS + s) * ept

    def body(j, carry):
        off = base + j * EB
        pltpu.sync_copy(src_hbm.at[pl.ds(off, EB)], idxs_v)
        pltpu.sync_copy(dst_hbm.at[pl.ds(off, EB)], idxd_v)
        pltpu.async_copy(m_hbm.at[idxs_v], rows_v, sem).wait()
        pltpu.sync_copy(rows_v, acc_sh.at[idxd_v], add=True)
        return carry
    lax.fori_loop(0, ept // EB, body, 0)
    plsc.subcore_barrier()

    for q in range(rpt // 128):
        r = row0 + q * 128
        pltpu.sync_copy(acc_sh.at[pl.ds(r, 128)], zbuf_v)
        pltpu.sync_copy(zbuf_v, out_hbm.at[c, pl.ds(r, 128)])


def _sc_deg(dst, npad):
    fn = pl.kernel(
        _sc_deg_body,
        out_type=jax.ShapeDtypeStruct((NW, npad), F32),
        mesh=_mesh(),
        compiler_params=pltpu.CompilerParams(needs_layout_passes=False),
        scratch_types=[
            pltpu.VMEM((EB,), jnp.int32),
            pltpu.VMEM((npad,), F32),
        ],
    )
    return fn(dst)


def _sc_agg(m, src, dst, h):
    npad = m.shape[0]
    fn = pl.kernel(
        _sc_agg_body,
        out_type=jax.ShapeDtypeStruct((NC, npad, h), F32),
        mesh=_mesh(),
        compiler_params=pltpu.CompilerParams(needs_layout_passes=False),
        scratch_types=[
            pltpu.VMEM((EB,), jnp.int32),
            pltpu.VMEM((EB,), jnp.int32),
            pltpu.VMEM((EB, 128), F32),
            pltpu.VMEM((128, 128), F32),
            pltpu.SemaphoreType.DMA,
            pltpu.VMEM_SHARED((npad, 128), F32),
        ],
    )
    return fn(m, src, dst)


def _tc0_body(degp_ref, x_ref, w_ref, dinv_ref, m_ref):
    deg = jnp.sum(degp_ref[...], axis=0) + 1.0
    dinv = lax.rsqrt(deg)[:, None]
    dinv_ref[...] = dinv
    t = jnp.dot(x_ref[...], w_ref[...], precision=HIGH,
                preferred_element_type=F32)
    m_ref[...] = t * dinv


def _tc_layer_body(p_ref, m_ref, dinv_ref, b_ref, w_ref, out_ref):
    dinv = dinv_ref[...]
    h = jnp.maximum((p_ref[0] + p_ref[1] + m_ref[...]) * dinv + b_ref[...],
                    0.0)
    t = jnp.dot(h, w_ref[...], precision=HIGH, preferred_element_type=F32)
    out_ref[...] = t * dinv


def _tc_final_body(p_ref, m_ref, dinv_ref, b_ref, batch_ref, wl_ref, bl_ref,
                   out_ref, pool_s, cnt_s):
    i = pl.program_id(0)

    @pl.when(i == 0)
    def _init():
        pool_s[...] = jnp.zeros_like(pool_s)
        cnt_s[...] = jnp.zeros_like(cnt_s)

    dinv = dinv_ref[...]
    h3 = (p_ref[0] + p_ref[1] + m_ref[...]) * dinv + b_ref[...]
    g = lax.broadcasted_iota(jnp.int32, (1, 128), 1)
    oh = (batch_ref[...] == g).astype(F32)
    pool_s[...] += lax.dot_general(oh, h3, (((0,), (0,)), ((), ())),
                                   precision=HIGH, preferred_element_type=F32)
    cnt_s[...] += jnp.sum(oh, axis=0)[None, :]

    @pl.when(i == pl.num_programs(0) - 1)
    def _fin():
        cnt = jnp.maximum(cnt_s[...], 1.0)
        pooled = pool_s[...][:64] / cnt[0, :64][:, None]
        out_ref[...] = jnp.dot(pooled, wl_ref[...], precision=HIGH,
                               preferred_element_type=F32) + bl_ref[...]


def _tc0(degp, xp, w1):
    npad, f = xp.shape
    h = w1.shape[1]
    return pl.pallas_call(
        _tc0_body,
        grid=(npad // RB,),
        in_specs=[
            pl.BlockSpec((NW, RB), lambda i: (0, i)),
            pl.BlockSpec((RB, f), lambda i: (i, 0)),
            pl.BlockSpec((f, h), lambda i: (0, 0)),
        ],
        out_specs=[
            pl.BlockSpec((RB, 1), lambda i: (i, 0)),
            pl.BlockSpec((RB, h), lambda i: (i, 0)),
        ],
        out_shape=[
            jax.ShapeDtypeStruct((npad, 1), F32),
            jax.ShapeDtypeStruct((npad, h), F32),
        ],
    )(degp, xp, w1)


def _tc_layer(p, m, dinv, b, w):
    npad, h = m.shape
    return pl.pallas_call(
        _tc_layer_body,
        grid=(npad // RB,),
        in_specs=[
            pl.BlockSpec((NC, RB, h), lambda i: (0, i, 0)),
            pl.BlockSpec((RB, h), lambda i: (i, 0)),
            pl.BlockSpec((RB, 1), lambda i: (i, 0)),
            pl.BlockSpec((1, h), lambda i: (0, 0)),
            pl.BlockSpec((h, h), lambda i: (0, 0)),
        ],
        out_specs=pl.BlockSpec((RB, h), lambda i: (i, 0)),
        out_shape=jax.ShapeDtypeStruct((npad, h), F32),
    )(p, m, dinv, b, w)


def _tc_final(p, m, dinv, b, batch2d, wlp, blp):
    npad, h = m.shape
    return pl.pallas_call(
        _tc_final_body,
        grid=(npad // RB,),
        in_specs=[
            pl.BlockSpec((NC, RB, h), lambda i: (0, i, 0)),
            pl.BlockSpec((RB, h), lambda i: (i, 0)),
            pl.BlockSpec((RB, 1), lambda i: (i, 0)),
            pl.BlockSpec((1, h), lambda i: (0, 0)),
            pl.BlockSpec((RB, 1), lambda i: (i, 0)),
            pl.BlockSpec((h, 128), lambda i: (0, 0)),
            pl.BlockSpec((1, 128), lambda i: (0, 0)),
        ],
        out_specs=pl.BlockSpec((64, 128), lambda i: (0, 0)),
        out_shape=jax.ShapeDtypeStruct((64, 128), F32),
        scratch_shapes=[
            pltpu.VMEM((128, 128), F32),
            pltpu.VMEM((1, 128), F32),
        ],
    )(p, m, dinv, b, batch2d, wlp, blp)


def kernel(x, edge_index, batch, W1, b1, W2, b2, W3, b3, Wl, bl):
    n, f = x.shape
    h = W1.shape[1]
    c = Wl.shape[1]
    e = edge_index.shape[1]
    assert e % (NW * EB) == 0
    npad = ((n + RB - 1) // RB) * RB

    xp = jnp.pad(x, ((0, npad - n), (0, 0)))
    batch2d = jnp.pad(batch, (0, npad - n), constant_values=127)[:, None]
    b1r = b1[None, :]
    b2r = b2[None, :]
    b3r = b3[None, :]
    wlp = jnp.pad(Wl, ((0, 0), (0, 128 - c)))
    blp = jnp.pad(bl, (0, 128 - c))[None, :]

    src = edge_index[0]
    dst = edge_index[1]
    degp = _sc_deg(dst, npad)
    dinv, m1 = _tc0(degp, xp, W1)
    p1 = _sc_agg(m1, src, dst, h)
    m2 = _tc_layer(p1, m1, dinv, b1r, W2)
    p2 = _sc_agg(m2, src, dst, h)
    m3 = _tc_layer(p2, m2, dinv, b2r, W3)
    p3 = _sc_agg(m3, src, dst, h)
    out = _tc_final(p3, m3, dinv, b3r, batch2d, wlp, blp)
    return out[:, :c]
